# Initial kernel scaffold; baseline (speedup 1.0000x reference)
import jax
import jax.numpy as jnp
from jax.experimental import pallas as pl


def kernel(x, edge_index, edge_attr, batch, W1, We, be, Wn1, bn1, Wn2, bn2, Wfc, bfc):
    raise NotImplementedError("write your pallas kernel here")


# speedup vs baseline: 1.9711x; 1.9711x over previous
import functools

import jax
import jax.numpy as jnp
from jax import lax
from jax.experimental import pallas as pl
from jax.experimental.pallas import tpu as pltpu
from jax.experimental.pallas import tpu_sc as plsc

_N = 10000
_E = 320000
_D = 128
_DE = 16
_G = 64
_NC = 2
_NS = 16
_CH = 128
_CPT = 160
_EPT = _CH * _CPT
_EP = _NS * _EPT
_NCH = _E // _CH
_HN = 5000
_HR = 5248
_RPT = _HR // _NS
_BN = 1000
_NB = _N // _BN
_NBH = _HN // _BN
_BE = 12800
_NBE = _E // _BE

_mesh = plsc.VectorSubcoreMesh(core_axis_name="c", subcore_axis_name="s")


def _sc_body(x_hbm, src_hbm, dst_hbm, z128_hbm, acc_out,
             dst_v, src_c, rows, sgs, sis, acc_sh, em_ops):
    c = lax.axis_index("c")
    s = lax.axis_index("s")
    pltpu.sync_copy(dst_hbm.at[c, s], dst_v)
    r0 = s * _RPT
    pltpu.sync_copy(z128_hbm, acc_sh.at[pl.ds(r0, _RPT)])
    plsc.subcore_barrier()

    em_start, em_finish = em_ops

    def idx_start(j, q):
        pltpu.async_copy(src_hbm.at[s, j], src_c.at[q], sis[q])

    def idx_wait(j, q):
        pltpu.make_async_copy(src_hbm.at[s, j], src_c.at[q], sis[q]).wait()

    def gather_start(j, q, b):
        pltpu.async_copy(x_hbm.at[src_c.at[q]], rows[b], sgs[b])

    def gather_wait(j, q, b):
        pltpu.make_async_copy(x_hbm.at[src_c.at[q]], rows[b], sgs[b]).wait()

    for q in range(4):
        idx_start(q, q)
    for b in range(2):
        idx_wait(b, b)
        gather_start(b, b, b)
        em_start(b, b)

    def body(i, carry):
        for b in range(4):
            j = 4 * i + b
            gather_wait(j, b, b % 2)
            pltpu.sync_copy(rows[b % 2], acc_sh.at[dst_v.at[j]], add=True)
            em_finish(j, b % 2)

            @pl.when(j + 2 < _CPT)
            def _():
                idx_wait(j + 2, (b + 2) % 4)
                gather_start(j + 2, (b + 2) % 4, b % 2)
                em_start(j + 2, b % 2)

            @pl.when(j + 4 < _CPT)
            def _():
                idx_start(j + 4, b)

        return carry

    lax.fori_loop(0, _CPT // 4, body, 0)

    plsc.subcore_barrier()
    pltpu.sync_copy(acc_sh.at[pl.ds(r0, _RPT)], acc_out.at[c, pl.ds(r0, _RPT)])


@functools.partial(
    pl.kernel,
    out_type=[jax.ShapeDtypeStruct((_NC, _HR, _D), jnp.float32)],
    mesh=_mesh,
    scratch_types=[
        pltpu.VMEM((_CPT, _CH), jnp.int32),
        pltpu.VMEM((4, _CH), jnp.int32),
        pltpu.VMEM((_CH, _D), jnp.float32),
        pltpu.VMEM((_CH, _D), jnp.float32),
        pltpu.VMEM_SHARED((_HR, _D), jnp.float32),
        pltpu.SemaphoreType.DMA,
        pltpu.SemaphoreType.DMA,
        pltpu.SemaphoreType.DMA,
        pltpu.SemaphoreType.DMA,
        pltpu.SemaphoreType.DMA,
        pltpu.SemaphoreType.DMA,
    ],
)
def _sc_edge_pass_a(x_hbm, src_hbm, dst_hbm, z128_hbm,
                    acc_out,
                    dst_v, src_c, rows0, rows1,
                    acc_sh, sg0, sg1, si0, si1, si2, si3):
    def em_nop(j, b):
        del j, b

    _sc_body(x_hbm, src_hbm, dst_hbm, z128_hbm, acc_out,
             dst_v, src_c, (rows0, rows1), (sg0, sg1), (si0, si1, si2, si3),
             acc_sh, (em_nop, em_nop))


@functools.partial(
    pl.kernel,
    out_type=[jax.ShapeDtypeStruct((_NC, _HR, _D), jnp.float32)],
    mesh=_mesh,
    scratch_types=[
        pltpu.VMEM((_CPT, _CH), jnp.int32),
        pltpu.VMEM((4, _CH), jnp.int32),
        pltpu.VMEM((_CH, _D), jnp.float32),
        pltpu.VMEM((_CH, _D), jnp.float32),
        pltpu.VMEM((_CH, _D), jnp.float32),
        pltpu.VMEM((_CH, _D), jnp.float32),
        pltpu.VMEM_SHARED((_HR, _D), jnp.float32),
        pltpu.SemaphoreType.DMA,
        pltpu.SemaphoreType.DMA,
        pltpu.SemaphoreType.DMA,
        pltpu.SemaphoreType.DMA,
        pltpu.SemaphoreType.DMA,
        pltpu.SemaphoreType.DMA,
        pltpu.SemaphoreType.DMA,
        pltpu.SemaphoreType.DMA,
    ],
)
def _sc_edge_pass_b(x_hbm, src_hbm, dst_hbm, em_hbm, z128_hbm,
                    acc_out,
                    dst_v, src_c, rows0, rows1, em0, em1,
                    acc_sh, sg0, sg1, sm0, sm1, si0, si1, si2, si3):
    s = lax.axis_index("s")
    emb = (em0, em1)
    sms = (sm0, sm1)

    def em_start(j, b):
        g = s * _CPT + j

        @pl.when(g < _NCH)
        def _():
            pltpu.async_copy(em_hbm.at[g], emb[b], sms[b])

    def em_finish(j, b):
        g = s * _CPT + j

        @pl.when(g < _NCH)
        def _():
            pltpu.make_async_copy(em_hbm.at[g], emb[b], sms[b]).wait()
            pltpu.sync_copy(emb[b], acc_sh.at[dst_v.at[j]], add=True)

    def inner(dv, sc_, r0_, r1_, acc, sg0_, sg1_, si_):
        pass

    _sc_body(x_hbm, src_hbm, dst_hbm, z128_hbm, acc_out,
             dst_v, src_c, (rows0, rows1), (sg0, sg1), (si0, si1, si2, si3),
             acc_sh, (em_start, em_finish))


def _tc_em_body(ea_ref, we_ref, be_ref, em_ref):
    em_ref[...] = lax.dot_general(
        ea_ref[...], we_ref[...], (((0,), (0,)), ((), ())),
        preferred_element_type=jnp.float32) + be_ref[...]


def _tc_pool_stats_body(p_ref, x_ref, b_ref, vmsg_ref, cnt_ref, accv, accc):
    i = pl.program_id(0)

    @pl.when(i == 0)
    def _():
        accv[...] = jnp.zeros_like(accv)
        accc[...] = jnp.zeros_like(accc)

    bvec = b_ref[0, 0, :]
    m = (lax.broadcasted_iota(jnp.int32, (_G, _BN), 0) == bvec[None, :]).astype(jnp.float32)
    rows = p_ref[0] + x_ref[...]
    accv[...] += jnp.dot(m, rows, preferred_element_type=jnp.float32)
    accc[...] += jnp.sum(m, axis=1, keepdims=True)

    @pl.when(i == _NB - 1)
    def _():
        cnt = jnp.maximum(accc[...], 1.0)
        cnt_ref[...] = cnt
        vmsg_ref[...] = accv[...] / cnt


def _tc_layer1_body(p_ref, x_ref, b_ref, vmsg_ref, w1_ref, x1_ref):
    bvec = b_ref[0, 0, :]
    mt = (bvec[:, None] == lax.broadcasted_iota(jnp.int32, (_BN, _G), 1)).astype(jnp.float32)
    vm = jnp.dot(mt, vmsg_ref[...], preferred_element_type=jnp.float32)
    t = p_ref[0] + x_ref[...] + vm
    x1_ref[...] = jax.nn.relu(jnp.dot(t, w1_ref[...], preferred_element_type=jnp.float32)) + x_ref[...]


def _tc_layer2_body(q_ref, x1_ref, b_ref, cnt_ref,
                    wn1_ref, bn1_ref, wn2_ref, bn2_ref,
                    wfc_ref, bfc_ref, y_ref, accp):
    i = pl.program_id(0)

    @pl.when(i == 0)
    def _():
        accp[...] = jnp.zeros_like(accp)

    agg = q_ref[0]
    h = jax.nn.relu(jnp.dot(agg, wn1_ref[...], preferred_element_type=jnp.float32) + bn1_ref[...])
    out2 = jnp.dot(h, wn2_ref[...], preferred_element_type=jnp.float32) + bn2_ref[...]
    x2 = out2 + x1_ref[...]
    bvec = b_ref[0, 0, :]
    m = (lax.broadcasted_iota(jnp.int32, (_G, _BN), 0) == bvec[None, :]).astype(jnp.float32)
    accp[...] += jnp.dot(m, x2, preferred_element_type=jnp.float32)

    @pl.when(i == _NB - 1)
    def _():
        pooled = accp[...] / cnt_ref[...]
        y_ref[...] = jnp.dot(pooled, wfc_ref[...], preferred_element_type=jnp.float32) + bfc_ref[...]


def _full(shape):
    return pl.BlockSpec(shape, lambda i: tuple(0 for _ in shape))


def _half_spec(minor):
    return pl.BlockSpec((1, _BN, minor), lambda i: (i // _NBH, i % _NBH, 0))


_tc_em = pl.pallas_call(
    _tc_em_body,
    grid=(_NBE,),
    in_specs=[
        pl.BlockSpec((_DE, _BE), lambda i: (0, i)),
        _full((_DE, _D)),
        _full((1, _D)),
    ],
    out_specs=pl.BlockSpec((_BE, _D), lambda i: (i, 0)),
    out_shape=jax.ShapeDtypeStruct((_E, _D), jnp.float32),
)

_tc_pool_stats = pl.pallas_call(
    _tc_pool_stats_body,
    grid=(_NB,),
    in_specs=[
        _half_spec(_D),
        pl.BlockSpec((_BN, _D), lambda i: (i, 0)),
        pl.BlockSpec((1, 1, _BN), lambda i: (i, 0, 0)),
    ],
    out_specs=[_full((_G, _D)), _full((_G, 1))],
    out_shape=[
        jax.ShapeDtypeStruct((_G, _D), jnp.float32),
        jax.ShapeDtypeStruct((_G, 1), jnp.float32),
    ],
    scratch_shapes=[
        pltpu.VMEM((_G, _D), jnp.float32),
        pltpu.VMEM((_G, 1), jnp.float32),
    ],
)

_tc_layer1 = pl.pallas_call(
    _tc_layer1_body,
    grid=(_NB,),
    in_specs=[
        _half_spec(_D),
        pl.BlockSpec((_BN, _D), lambda i: (i, 0)),
        pl.BlockSpec((1, 1, _BN), lambda i: (i, 0, 0)),
        _full((_G, _D)),
        _full((_D, _D)),
    ],
    out_specs=pl.BlockSpec((_BN, _D), lambda i: (i, 0)),
    out_shape=jax.ShapeDtypeStruct((_N, _D), jnp.float32),
)

_tc_layer2 = pl.pallas_call(
    _tc_layer2_body,
    grid=(_NB,),
    in_specs=[
        _half_spec(_D),
        pl.BlockSpec((_BN, _D), lambda i: (i, 0)),
        pl.BlockSpec((1, 1, _BN), lambda i: (i, 0, 0)),
        _full((_G, 1)),
        _full((_D, _D)),
        _full((1, _D)),
        _full((_D, _D)),
        _full((1, _D)),
        _full((_D, _D)),
        _full((1, _D)),
    ],
    out_specs=_full((_G, _D)),
    out_shape=jax.ShapeDtypeStruct((_G, _D), jnp.float32),
    scratch_shapes=[pltpu.VMEM((_G, _D), jnp.float32)],
)


def kernel(x, edge_index, edge_attr, batch, W1, We, be, Wn1, bn1, Wn2, bn2, Wfc, bfc):
    src = edge_index[0]
    dst = edge_index[1]
    padn = _EP - _E
    pad_src = jnp.zeros((padn,), jnp.int32)
    srcp = jnp.concatenate([src, pad_src]).reshape(_NS, _CPT, _CH)
    dstf = jnp.concatenate([dst, jnp.full((padn,), jnp.int32(1 << 24))])
    dummy = _HN + (jnp.arange(_EP, dtype=jnp.int32) % (_HR - _HN))
    dstp = jnp.stack([
        jnp.where((dstf >= c * _HN) & (dstf < (c + 1) * _HN), dstf - c * _HN, dummy)
        for c in range(_NC)
    ]).reshape(_NC, _NS, _CPT, _CH)
    z128 = jnp.zeros((_RPT, _D), jnp.float32)
    batchr = batch.reshape(_NB, 1, _BN)

    (p,) = _sc_edge_pass_a(x, srcp, dstp, z128)
    vmsg, cnt = _tc_pool_stats(p, x, batchr)
    x1 = _tc_layer1(p, x, batchr, vmsg, W1)
    em = _tc_em(edge_attr.T, We, be.reshape(1, _D)).reshape(_NCH, _CH, _D)
    (q,) = _sc_edge_pass_b(x1, srcp, dstp, em, z128)

    y = _tc_layer2(q, x1, batchr, cnt,
                   Wn1, bn1.reshape(1, _D),
                   Wn2, bn2.reshape(1, _D), Wfc, bfc.reshape(1, _D))
    return y


# speedup vs baseline: 3.1463x; 1.5963x over previous
import functools

import jax
import jax.numpy as jnp
from jax import lax
from jax.experimental import pallas as pl
from jax.experimental.pallas import tpu as pltpu
from jax.experimental.pallas import tpu_sc as plsc

_N = 10000
_E = 320000
_D = 128
_DE = 16
_G = 64
_NC = 2
_NS = 16
_NW = _NC * _NS
_EPT = 10240
_EP = _NW * _EPT
_CHA = 128
_CPA = _EPT // _CHA
_CHB = 64
_CPB = _EPT // _CHB
_NCHB = _E // _CHB
_RA = 10112
_RPT = _RA // _NS
_BN = 1000
_NB = _N // _BN
_BE = 12800
_NBE = _E // _BE

_mesh = plsc.VectorSubcoreMesh(core_axis_name="c", subcore_axis_name="s")


def _sc_body(x_hbm, src_hbm, dst_hbm, z128_hbm, acc_out, w, cpt,
             dst_c, src_c, rows, sgs, sis, acc_sh, em_ops):
    c = lax.axis_index("c")
    s = lax.axis_index("s")
    r0 = s * _RPT
    pltpu.sync_copy(z128_hbm, acc_sh.at[pl.ds(r0, _RPT)])
    plsc.subcore_barrier()

    em_start, em_finish = em_ops

    def idx_start(j, q):
        pltpu.async_copy(src_hbm.at[w, j], src_c.at[q], sis[q])
        pltpu.async_copy(dst_hbm.at[w, j], dst_c.at[q], sis[q])

    def idx_wait(j, q):
        pltpu.make_async_copy(src_hbm.at[w, j], src_c.at[q], sis[q]).wait()
        pltpu.make_async_copy(dst_hbm.at[w, j], dst_c.at[q], sis[q]).wait()

    def gather_start(j, q, b):
        pltpu.async_copy(x_hbm.at[src_c.at[q]], rows[b], sgs[b])

    def gather_wait(j, q, b):
        pltpu.make_async_copy(x_hbm.at[src_c.at[q]], rows[b], sgs[b]).wait()

    for q in range(4):
        idx_start(q, q)
    for b in range(2):
        idx_wait(b, b)
        gather_start(b, b, b)
        em_start(b, b)

    def body(i, carry):
        for b in range(4):
            j = 4 * i + b
            gather_wait(j, b, b % 2)
            pltpu.sync_copy(rows[b % 2], acc_sh.at[dst_c.at[b]], add=True)
            em_finish(j, b % 2, b)

            @pl.when(j + 2 < cpt)
            def _():
                idx_wait(j + 2, (b + 2) % 4)
                gather_start(j + 2, (b + 2) % 4, b % 2)
                em_start(j + 2, b % 2)

            @pl.when(j + 4 < cpt)
            def _():
                idx_start(j + 4, b)

        return carry

    lax.fori_loop(0, cpt // 4, body, 0)

    plsc.subcore_barrier()
    pltpu.sync_copy(acc_sh.at[pl.ds(r0, _RPT)], acc_out.at[c, pl.ds(r0, _RPT)])


@functools.partial(
    pl.kernel,
    out_type=[jax.ShapeDtypeStruct((_NC, _RA, _D), jnp.float32)],
    mesh=_mesh,
    scratch_types=[
        pltpu.VMEM((4, _CHA), jnp.int32),
        pltpu.VMEM((4, _CHA), jnp.int32),
        pltpu.VMEM((_CHA, _D), jnp.float32),
        pltpu.VMEM((_CHA, _D), jnp.float32),
        pltpu.VMEM_SHARED((_RA, _D), jnp.float32),
        pltpu.SemaphoreType.DMA,
        pltpu.SemaphoreType.DMA,
        pltpu.SemaphoreType.DMA,
        pltpu.SemaphoreType.DMA,
        pltpu.SemaphoreType.DMA,
        pltpu.SemaphoreType.DMA,
    ],
)
def _sc_edge_pass_a(x_hbm, src_hbm, dst_hbm, z128_hbm,
                    acc_out,
                    dst_c, src_c, rows0, rows1,
                    acc_sh, sg0, sg1, si0, si1, si2, si3):
    c = lax.axis_index("c")
    s = lax.axis_index("s")
    w = c * _NS + s

    def em_nop(j, b, q=0):
        del j, b, q

    _sc_body(x_hbm, src_hbm, dst_hbm, z128_hbm, acc_out, w, _CPA,
             dst_c, src_c, (rows0, rows1), (sg0, sg1), (si0, si1, si2, si3),
             acc_sh, (em_nop, em_nop))


@functools.partial(
    pl.kernel,
    out_type=[jax.ShapeDtypeStruct((_NC, _RA, _D), jnp.float32)],
    mesh=_mesh,
    scratch_types=[
        pltpu.VMEM((4, _CHB), jnp.int32),
        pltpu.VMEM((4, _CHB), jnp.int32),
        pltpu.VMEM((_CHB, _D), jnp.float32),
        pltpu.VMEM((_CHB, _D), jnp.float32),
        pltpu.VMEM((_CHB, _D), jnp.float32),
        pltpu.VMEM((_CHB, _D), jnp.float32),
        pltpu.VMEM_SHARED((_RA, _D), jnp.float32),
        pltpu.SemaphoreType.DMA,
        pltpu.SemaphoreType.DMA,
        pltpu.SemaphoreType.DMA,
        pltpu.SemaphoreType.DMA,
        pltpu.SemaphoreType.DMA,
        pltpu.SemaphoreType.DMA,
        pltpu.SemaphoreType.DMA,
        pltpu.SemaphoreType.DMA,
    ],
)
def _sc_edge_pass_b(x_hbm, src_hbm, dst_hbm, em_hbm, z128_hbm,
                    acc_out,
                    dst_c, src_c, rows0, rows1, em0, em1,
                    acc_sh, sg0, sg1, sm0, sm1, si0, si1, si2, si3):
    c = lax.axis_index("c")
    s = lax.axis_index("s")
    w = c * _NS + s
    emb = (em0, em1)
    sms = (sm0, sm1)

    def em_start(j, b):
        g = w * _CPB + j

        @pl.when(g < _NCHB)
        def _():
            pltpu.async_copy(em_hbm.at[g], emb[b], sms[b])

    def em_finish(j, b, q):
        g = w * _CPB + j

        @pl.when(g < _NCHB)
        def _():
            pltpu.make_async_copy(em_hbm.at[g], emb[b], sms[b]).wait()
            pltpu.sync_copy(emb[b], acc_sh.at[dst_c.at[q]], add=True)

    _sc_body(x_hbm, src_hbm, dst_hbm, z128_hbm, acc_out, w, _CPB,
             dst_c, src_c, (rows0, rows1), (sg0, sg1), (si0, si1, si2, si3),
             acc_sh, (em_start, em_finish))


def _tc_em_body(ea_ref, we_ref, be_ref, em_ref):
    em_ref[...] = lax.dot_general(
        ea_ref[...], we_ref[...], (((0,), (0,)), ((), ())),
        preferred_element_type=jnp.float32) + be_ref[...]


def _tc_pool_stats_body(p_ref, x_ref, b_ref, vmsg_ref, cnt_ref, accv, accc):
    i = pl.program_id(0)

    @pl.when(i == 0)
    def _():
        accv[...] = jnp.zeros_like(accv)
        accc[...] = jnp.zeros_like(accc)

    bvec = b_ref[0, 0, :]
    m = (lax.broadcasted_iota(jnp.int32, (_G, _BN), 0) == bvec[None, :]).astype(jnp.float32)
    rows = p_ref[0] + p_ref[1] + x_ref[...]
    accv[...] += jnp.dot(m, rows, preferred_element_type=jnp.float32)
    accc[...] += jnp.sum(m, axis=1, keepdims=True)

    @pl.when(i == _NB - 1)
    def _():
        cnt = jnp.maximum(accc[...], 1.0)
        cnt_ref[...] = cnt
        vmsg_ref[...] = accv[...] / cnt


def _tc_layer1_body(p_ref, x_ref, b_ref, vmsg_ref, w1_ref, x1_ref):
    bvec = b_ref[0, 0, :]
    mt = (bvec[:, None] == lax.broadcasted_iota(jnp.int32, (_BN, _G), 1)).astype(jnp.float32)
    vm = jnp.dot(mt, vmsg_ref[...], preferred_element_type=jnp.float32)
    t = p_ref[0] + p_ref[1] + x_ref[...] + vm
    x1_ref[...] = jax.nn.relu(jnp.dot(t, w1_ref[...], preferred_element_type=jnp.float32)) + x_ref[...]


def _tc_layer2_body(q_ref, x1_ref, b_ref, cnt_ref,
                    wn1_ref, bn1_ref, wn2_ref, bn2_ref,
                    wfc_ref, bfc_ref, y_ref, accp):
    i = pl.program_id(0)

    @pl.when(i == 0)
    def _():
        accp[...] = jnp.zeros_like(accp)

    agg = q_ref[0] + q_ref[1]
    h = jax.nn.relu(jnp.dot(agg, wn1_ref[...], preferred_element_type=jnp.float32) + bn1_ref[...])
    out2 = jnp.dot(h, wn2_ref[...], preferred_element_type=jnp.float32) + bn2_ref[...]
    x2 = out2 + x1_ref[...]
    bvec = b_ref[0, 0, :]
    m = (lax.broadcasted_iota(jnp.int32, (_G, _BN), 0) == bvec[None, :]).astype(jnp.float32)
    accp[...] += jnp.dot(m, x2, preferred_element_type=jnp.float32)

    @pl.when(i == _NB - 1)
    def _():
        pooled = accp[...] / cnt_ref[...]
        y_ref[...] = jnp.dot(pooled, wfc_ref[...], preferred_element_type=jnp.float32) + bfc_ref[...]


def _full(shape):
    return pl.BlockSpec(shape, lambda i: tuple(0 for _ in shape))


def _part_spec(minor):
    return pl.BlockSpec((_NC, _BN, minor), lambda i: (0, i, 0))


_tc_em = pl.pallas_call(
    _tc_em_body,
    grid=(_NBE,),
    in_specs=[
        pl.BlockSpec((_DE, _BE), lambda i: (0, i)),
        _full((_DE, _D)),
        _full((1, _D)),
    ],
    out_specs=pl.BlockSpec((_BE, _D), lambda i: (i, 0)),
    out_shape=jax.ShapeDtypeStruct((_E, _D), jnp.float32),
)

_tc_pool_stats = pl.pallas_call(
    _tc_pool_stats_body,
    grid=(_NB,),
    in_specs=[
        _part_spec(_D),
        pl.BlockSpec((_BN, _D), lambda i: (i, 0)),
        pl.BlockSpec((1, 1, _BN), lambda i: (i, 0, 0)),
    ],
    out_specs=[_full((_G, _D)), _full((_G, 1))],
    out_shape=[
        jax.ShapeDtypeStruct((_G, _D), jnp.float32),
        jax.ShapeDtypeStruct((_G, 1), jnp.float32),
    ],
    scratch_shapes=[
        pltpu.VMEM((_G, _D), jnp.float32),
        pltpu.VMEM((_G, 1), jnp.float32),
    ],
)

_tc_layer1 = pl.pallas_call(
    _tc_layer1_body,
    grid=(_NB,),
    in_specs=[
        _part_spec(_D),
        pl.BlockSpec((_BN, _D), lambda i: (i, 0)),
        pl.BlockSpec((1, 1, _BN), lambda i: (i, 0, 0)),
        _full((_G, _D)),
        _full((_D, _D)),
    ],
    out_specs=pl.BlockSpec((_BN, _D), lambda i: (i, 0)),
    out_shape=jax.ShapeDtypeStruct((_N, _D), jnp.float32),
)

_tc_layer2 = pl.pallas_call(
    _tc_layer2_body,
    grid=(_NB,),
    in_specs=[
        _part_spec(_D),
        pl.BlockSpec((_BN, _D), lambda i: (i, 0)),
        pl.BlockSpec((1, 1, _BN), lambda i: (i, 0, 0)),
        _full((_G, 1)),
        _full((_D, _D)),
        _full((1, _D)),
        _full((_D, _D)),
        _full((1, _D)),
        _full((_D, _D)),
        _full((1, _D)),
    ],
    out_specs=_full((_G, _D)),
    out_shape=jax.ShapeDtypeStruct((_G, _D), jnp.float32),
    scratch_shapes=[pltpu.VMEM((_G, _D), jnp.float32)],
)


def kernel(x, edge_index, edge_attr, batch, W1, We, be, Wn1, bn1, Wn2, bn2, Wfc, bfc):
    src = edge_index[0]
    dst = edge_index[1]
    padn = _EP - _E
    pad_src = jnp.zeros((padn,), jnp.int32)
    pad_dst = _N + (jnp.arange(padn, dtype=jnp.int32) % (_RA - _N))
    srcf = jnp.concatenate([src, pad_src])
    dstf = jnp.concatenate([dst, pad_dst])
    srca = srcf.reshape(_NW, _CPA, _CHA)
    dsta = dstf.reshape(_NW, _CPA, _CHA)
    srcb = srcf.reshape(_NW, _CPB, _CHB)
    dstb = dstf.reshape(_NW, _CPB, _CHB)
    z128 = jnp.zeros((_RPT, _D), jnp.float32)
    batchr = batch.reshape(_NB, 1, _BN)

    (p,) = _sc_edge_pass_a(x, srca, dsta, z128)
    vmsg, cnt = _tc_pool_stats(p, x, batchr)
    x1 = _tc_layer1(p, x, batchr, vmsg, W1)
    em = _tc_em(edge_attr.T, We, be.reshape(1, _D)).reshape(_NCHB, _CHB, _D)
    (q,) = _sc_edge_pass_b(x1, srcb, dstb, em, z128)

    y = _tc_layer2(q, x1, batchr, cnt,
                   Wn1, bn1.reshape(1, _D),
                   Wn2, bn2.reshape(1, _D), Wfc, bfc.reshape(1, _D))
    return y


# speedup vs baseline: 3.3355x; 1.0601x over previous
import functools

import jax
import jax.numpy as jnp
from jax import lax
from jax.experimental import pallas as pl
from jax.experimental.pallas import tpu as pltpu
from jax.experimental.pallas import tpu_sc as plsc

_N = 10000
_E = 320000
_D = 128
_DE = 16
_G = 64
_NC = 2
_NS = 16
_NW = _NC * _NS
_EPT = 10240
_EP = _NW * _EPT
_CH = 64
_CPT = _EPT // _CH
_NCHB = _E // _CH
_RA = 10112
_RPT = _RA // _NS
_BN = 1000
_NB = _N // _BN
_BE = 12800
_NBE = _E // _BE

_mesh = plsc.VectorSubcoreMesh(core_axis_name="c", subcore_axis_name="s")


def _sc_body(x_hbm, src_hbm, dst_hbm, z128_hbm, acc_out, w,
             dst_c, src_c, rows, sgs, sss, sis, sds, acc_sh, em_ops):
    c = lax.axis_index("c")
    s = lax.axis_index("s")
    r0 = s * _RPT
    pltpu.sync_copy(z128_hbm, acc_sh.at[pl.ds(r0, _RPT)])
    plsc.subcore_barrier()

    em_start, em_finish = em_ops

    def gather_start(j, q, rb):
        pltpu.async_copy(x_hbm.at[src_c.at[q]], rows[rb], sgs[q])

    def gather_wait(j, q, rb):
        pltpu.make_async_copy(x_hbm.at[src_c.at[q]], rows[rb], sgs[q]).wait()

    def scatter_start(j, rb):
        pltpu.async_copy(rows[rb], acc_sh.at[dst_c.at[rb]], sss[rb], add=True)

    def scatter_wait(j, rb):
        pltpu.make_async_copy(rows[rb], acc_sh.at[dst_c.at[rb]],
                              sss[rb]).wait()

    for q in range(4):
        pltpu.sync_copy(dst_hbm.at[w, q], dst_c.at[q])
    for b in range(2):
        pltpu.sync_copy(src_hbm.at[w, b], src_c.at[b])
        gather_start(b, b, b)
        em_start(b, b)

    def body(i, carry):
        for b in range(4):
            j = 4 * i + b
            gather_wait(j, b % 2, b)
            scatter_start(j, b)
            em_finish(j, b % 2, b)

            @pl.when(j >= 2)
            def _():
                scatter_wait(j - 2, (b + 2) % 4)

            @pl.when(j + 2 < _CPT)
            def _():
                pltpu.async_copy(src_hbm.at[w, j + 2], src_c.at[b % 2],
                                 sis[b % 2])
                pltpu.async_copy(dst_hbm.at[w, j + 2], dst_c.at[(b + 2) % 4],
                                 sds[b % 2])
                pltpu.make_async_copy(src_hbm.at[w, j + 2], src_c.at[b % 2],
                                      sis[b % 2]).wait()
                pltpu.make_async_copy(dst_hbm.at[w, j + 2], dst_c.at[(b + 2) % 4],
                                      sds[b % 2]).wait()
                gather_start(j + 2, b % 2, (b + 2) % 4)
                em_start(j + 2, b % 2)

        return carry

    lax.fori_loop(0, _CPT // 4, body, 0)

    scatter_wait(_CPT - 2, (_CPT - 2) % 4)
    scatter_wait(_CPT - 1, (_CPT - 1) % 4)
    plsc.subcore_barrier()
    pltpu.sync_copy(acc_sh.at[pl.ds(r0, _RPT)], acc_out.at[c, pl.ds(r0, _RPT)])


@functools.partial(
    pl.kernel,
    out_type=[jax.ShapeDtypeStruct((_NC, _RA, _D), jnp.float32)],
    mesh=_mesh,
    scratch_types=[
        pltpu.VMEM((4, _CH), jnp.int32),
        pltpu.VMEM((2, _CH), jnp.int32),
        pltpu.VMEM((_CH, _D), jnp.float32),
        pltpu.VMEM((_CH, _D), jnp.float32),
        pltpu.VMEM((_CH, _D), jnp.float32),
        pltpu.VMEM((_CH, _D), jnp.float32),
        pltpu.VMEM_SHARED((_RA, _D), jnp.float32),
    ] + [pltpu.SemaphoreType.DMA] * 12,
)
def _sc_edge_pass_a(x_hbm, src_hbm, dst_hbm, z128_hbm,
                    acc_out,
                    dst_c, src_c, r0_, r1_, r2_, r3_,
                    acc_sh, sg0, sg1, sg2, sg3, ss0, ss1, ss2, ss3,
                    si0, si1, sd0, sd1):
    c = lax.axis_index("c")
    s = lax.axis_index("s")
    w = c * _NS + s

    def em_nop(j, b, q=0):
        del j, b, q

    _sc_body(x_hbm, src_hbm, dst_hbm, z128_hbm, acc_out, w,
             dst_c, src_c, (r0_, r1_, r2_, r3_), (sg0, sg1, sg2, sg3),
             (ss0, ss1, ss2, ss3), (si0, si1), (sd0, sd1),
             acc_sh, (em_nop, em_nop))


@functools.partial(
    pl.kernel,
    out_type=[jax.ShapeDtypeStruct((_NC, _RA, _D), jnp.float32)],
    mesh=_mesh,
    scratch_types=[
        pltpu.VMEM((4, _CH), jnp.int32),
        pltpu.VMEM((2, _CH), jnp.int32),
        pltpu.VMEM((_CH, _D), jnp.float32),
        pltpu.VMEM((_CH, _D), jnp.float32),
        pltpu.VMEM((_CH, _D), jnp.float32),
        pltpu.VMEM((_CH, _D), jnp.float32),
        pltpu.VMEM((_CH, _D), jnp.float32),
        pltpu.VMEM((_CH, _D), jnp.float32),
        pltpu.VMEM_SHARED((_RA, _D), jnp.float32),
    ] + [pltpu.SemaphoreType.DMA] * 14,
)
def _sc_edge_pass_b(x_hbm, src_hbm, dst_hbm, em_hbm, z128_hbm,
                    acc_out,
                    dst_c, src_c, r0_, r1_, r2_, r3_, em0, em1,
                    acc_sh, sg0, sg1, sg2, sg3, ss0, ss1, ss2, ss3,
                    si0, si1, sd0, sd1, sm0, sm1):
    c = lax.axis_index("c")
    s = lax.axis_index("s")
    w = c * _NS + s
    emb = (em0, em1)
    sms = (sm0, sm1)

    def em_start(j, b):
        g = w * _CPT + j

        @pl.when(g < _NCHB)
        def _():
            pltpu.async_copy(em_hbm.at[g], emb[b], sms[b])

    def em_finish(j, b, q):
        g = w * _CPT + j

        @pl.when(g < _NCHB)
        def _():
            pltpu.make_async_copy(em_hbm.at[g], emb[b], sms[b]).wait()
            pltpu.sync_copy(emb[b], acc_sh.at[dst_c.at[q]], add=True)

    _sc_body(x_hbm, src_hbm, dst_hbm, z128_hbm, acc_out, w,
             dst_c, src_c, (r0_, r1_, r2_, r3_), (sg0, sg1, sg2, sg3),
             (ss0, ss1, ss2, ss3), (si0, si1), (sd0, sd1),
             acc_sh, (em_start, em_finish))


def _tc_em_body(ea_ref, we_ref, be_ref, em_ref):
    em_ref[...] = lax.dot_general(
        ea_ref[...], we_ref[...], (((0,), (0,)), ((), ())),
        preferred_element_type=jnp.float32) + be_ref[...]


def _tc_pool_stats_body(p_ref, x_ref, b_ref, vmsg_ref, cnt_ref, accv, accc):
    i = pl.program_id(0)

    @pl.when(i == 0)
    def _():
        accv[...] = jnp.zeros_like(accv)
        accc[...] = jnp.zeros_like(accc)

    bvec = b_ref[0, 0, :]
    m = (lax.broadcasted_iota(jnp.int32, (_G, _BN), 0) == bvec[None, :]).astype(jnp.float32)
    rows = p_ref[0] + p_ref[1] + x_ref[...]
    accv[...] += jnp.dot(m, rows, preferred_element_type=jnp.float32)
    accc[...] += jnp.sum(m, axis=1, keepdims=True)

    @pl.when(i == _NB - 1)
    def _():
        cnt = jnp.maximum(accc[...], 1.0)
        cnt_ref[...] = cnt
        vmsg_ref[...] = accv[...] / cnt


def _tc_layer1_body(p_ref, x_ref, b_ref, vmsg_ref, w1_ref, x1_ref):
    bvec = b_ref[0, 0, :]
    mt = (bvec[:, None] == lax.broadcasted_iota(jnp.int32, (_BN, _G), 1)).astype(jnp.float32)
    vm = jnp.dot(mt, vmsg_ref[...], preferred_element_type=jnp.float32)
    t = p_ref[0] + p_ref[1] + x_ref[...] + vm
    x1_ref[...] = jax.nn.relu(jnp.dot(t, w1_ref[...], preferred_element_type=jnp.float32)) + x_ref[...]


def _tc_layer2_body(q_ref, x1_ref, b_ref, cnt_ref,
                    wn1_ref, bn1_ref, wn2_ref, bn2_ref,
                    wfc_ref, bfc_ref, y_ref, accp):
    i = pl.program_id(0)

    @pl.when(i == 0)
    def _():
        accp[...] = jnp.zeros_like(accp)

    agg = q_ref[0] + q_ref[1]
    h = jax.nn.relu(jnp.dot(agg, wn1_ref[...], preferred_element_type=jnp.float32) + bn1_ref[...])
    out2 = jnp.dot(h, wn2_ref[...], preferred_element_type=jnp.float32) + bn2_ref[...]
    x2 = out2 + x1_ref[...]
    bvec = b_ref[0, 0, :]
    m = (lax.broadcasted_iota(jnp.int32, (_G, _BN), 0) == bvec[None, :]).astype(jnp.float32)
    accp[...] += jnp.dot(m, x2, preferred_element_type=jnp.float32)

    @pl.when(i == _NB - 1)
    def _():
        pooled = accp[...] / cnt_ref[...]
        y_ref[...] = jnp.dot(pooled, wfc_ref[...], preferred_element_type=jnp.float32) + bfc_ref[...]


def _full(shape):
    return pl.BlockSpec(shape, lambda i: tuple(0 for _ in shape))


def _part_spec(minor):
    return pl.BlockSpec((_NC, _BN, minor), lambda i: (0, i, 0))


_tc_em = pl.pallas_call(
    _tc_em_body,
    grid=(_NBE,),
    in_specs=[
        pl.BlockSpec((_DE, _BE), lambda i: (0, i)),
        _full((_DE, _D)),
        _full((1, _D)),
    ],
    out_specs=pl.BlockSpec((_BE, _D), lambda i: (i, 0)),
    out_shape=jax.ShapeDtypeStruct((_E, _D), jnp.float32),
)

_tc_pool_stats = pl.pallas_call(
    _tc_pool_stats_body,
    grid=(_NB,),
    in_specs=[
        _part_spec(_D),
        pl.BlockSpec((_BN, _D), lambda i: (i, 0)),
        pl.BlockSpec((1, 1, _BN), lambda i: (i, 0, 0)),
    ],
    out_specs=[_full((_G, _D)), _full((_G, 1))],
    out_shape=[
        jax.ShapeDtypeStruct((_G, _D), jnp.float32),
        jax.ShapeDtypeStruct((_G, 1), jnp.float32),
    ],
    scratch_shapes=[
        pltpu.VMEM((_G, _D), jnp.float32),
        pltpu.VMEM((_G, 1), jnp.float32),
    ],
)

_tc_layer1 = pl.pallas_call(
    _tc_layer1_body,
    grid=(_NB,),
    in_specs=[
        _part_spec(_D),
        pl.BlockSpec((_BN, _D), lambda i: (i, 0)),
        pl.BlockSpec((1, 1, _BN), lambda i: (i, 0, 0)),
        _full((_G, _D)),
        _full((_D, _D)),
    ],
    out_specs=pl.BlockSpec((_BN, _D), lambda i: (i, 0)),
    out_shape=jax.ShapeDtypeStruct((_N, _D), jnp.float32),
)

_tc_layer2 = pl.pallas_call(
    _tc_layer2_body,
    grid=(_NB,),
    in_specs=[
        _part_spec(_D),
        pl.BlockSpec((_BN, _D), lambda i: (i, 0)),
        pl.BlockSpec((1, 1, _BN), lambda i: (i, 0, 0)),
        _full((_G, 1)),
        _full((_D, _D)),
        _full((1, _D)),
        _full((_D, _D)),
        _full((1, _D)),
        _full((_D, _D)),
        _full((1, _D)),
    ],
    out_specs=_full((_G, _D)),
    out_shape=jax.ShapeDtypeStruct((_G, _D), jnp.float32),
    scratch_shapes=[pltpu.VMEM((_G, _D), jnp.float32)],
)


def kernel(x, edge_index, edge_attr, batch, W1, We, be, Wn1, bn1, Wn2, bn2, Wfc, bfc):
    src = edge_index[0]
    dst = edge_index[1]
    padn = _EP - _E
    pad_src = jnp.zeros((padn,), jnp.int32)
    pad_dst = _N + (jnp.arange(padn, dtype=jnp.int32) % (_RA - _N))
    srcf = jnp.concatenate([src, pad_src])
    dstf = jnp.concatenate([dst, pad_dst])
    srca = srcf.reshape(_NW, _CPT, _CH)
    dsta = dstf.reshape(_NW, _CPT, _CH)
    z128 = jnp.zeros((_RPT, _D), jnp.float32)
    batchr = batch.reshape(_NB, 1, _BN)

    (p,) = _sc_edge_pass_a(x, srca, dsta, z128)
    vmsg, cnt = _tc_pool_stats(p, x, batchr)
    x1 = _tc_layer1(p, x, batchr, vmsg, W1)
    em = _tc_em(edge_attr.T, We, be.reshape(1, _D)).reshape(_NCHB, _CH, _D)
    (q,) = _sc_edge_pass_b(x1, srca, dsta, em, z128)

    y = _tc_layer2(q, x1, batchr, cnt,
                   Wn1, bn1.reshape(1, _D),
                   Wn2, bn2.reshape(1, _D), Wfc, bfc.reshape(1, _D))
    return y
